# baseline (speedup 1.0000x reference)
import functools

import jax
import jax.numpy as jnp
from jax import lax
from jax.experimental import pallas as pl
from jax.experimental.pallas import tpu as pltpu
from jax.experimental.pallas import tpu_sc as plsc

N_NODES = 10000
D = 128
DH = D // 2
N_PAD = 10240
NC, NS, L = 2, 16, 16
CHUNK = 128
ROWS_PER_TILE = N_PAD // NS
K = 8


def _mm_body(x_ref, w_ref, b_ref, h_ref):
    h = (
        jnp.dot(x_ref[...], w_ref[...], preferred_element_type=jnp.float32)
        + b_ref[...]
    )
    h_ref[0, ...] = h[:, :DH]
    h_ref[1, ...] = h[:, DH:]


def _matmul(x, W, b):
    M = x.shape[0]
    BM = 1024
    return pl.pallas_call(
        _mm_body,
        grid=(M // BM,),
        in_specs=[
            pl.BlockSpec((BM, D), lambda i: (i, 0)),
            pl.BlockSpec((D, D), lambda i: (0, 0)),
            pl.BlockSpec((1, D), lambda i: (0, 0)),
        ],
        out_specs=pl.BlockSpec((2, BM, DH), lambda i: (0, i, 0)),
        out_shape=jax.ShapeDtypeStruct((2, M, DH), jnp.float32),
    )(x, W, b.reshape(1, D))


def _agg_body(h_hbm, src_hbm, dst_hbm, ew_hbm, out_hbm,
              sidx_v, didx_v, ew_v, rows_v, srows_v, h_sh, acc_sh,
              sem_g, sem_s, sem_e, nblocks):
    c = lax.axis_index("c")
    s = lax.axis_index("s")

    zero = jnp.zeros((L,), jnp.float32)

    def _zrow(i, carry):
        for j in range(DH // L):
            rows_v[0, i, pl.ds(j * L, L)] = zero
        return carry

    lax.fori_loop(0, CHUNK, _zrow, 0)
    for k in range(ROWS_PER_TILE // CHUNK):
        r0 = s * ROWS_PER_TILE + k * CHUNK
        pltpu.sync_copy(rows_v.at[0], acc_sh.at[pl.ds(r0, CHUNK)])
        pltpu.sync_copy(h_hbm.at[c, pl.ds(r0, CHUNK)], rows_v.at[1])
        pltpu.sync_copy(rows_v.at[1], h_sh.at[pl.ds(r0, CHUNK)])
    plsc.subcore_barrier()

    def _ed_copy(b, eb):
        base = s * nblocks + b
        pltpu.async_copy(src_hbm.at[pl.ds(base * K, K)], sidx_v.at[eb], sem_e)
        pltpu.async_copy(dst_hbm.at[pl.ds(base * K, K)], didx_v.at[eb], sem_e)
        pltpu.async_copy(ew_hbm.at[pl.ds(base * K, K)], ew_v.at[eb], sem_e)

    def _ed_wait(eb):
        pltpu.make_async_copy(src_hbm.at[pl.ds(0, K)], sidx_v.at[eb], sem_e).wait()
        pltpu.make_async_copy(dst_hbm.at[pl.ds(0, K)], didx_v.at[eb], sem_e).wait()
        pltpu.make_async_copy(ew_hbm.at[pl.ds(0, K)], ew_v.at[eb], sem_e).wait()

    def _scale(buf, eb, g):
        rb = rows_v.at[buf]
        sb = srows_v.at[buf]

        def _grp(i, carry):
            w16 = ew_v[eb, g, pl.ds(i * L, L)]
            for ii in range(L):
                e = i * L + ii
                w = w16[ii]
                for j in range(DH // L):
                    sb[e, pl.ds(j * L, L)] = rb[e, pl.ds(j * L, L)] * w
            return carry

        lax.fori_loop(0, CHUNK // L, _grp, 0)

    def _iter(b, g, buf, eb, last):
        pltpu.make_async_copy(h_sh.at[sidx_v.at[eb, 0]],
                              rows_v.at[buf], sem_g).wait()

        if not last:
            pltpu.async_copy(h_sh.at[sidx_v.at[eb, g + 1]],
                             rows_v.at[1 - buf], sem_g)
        else:
            @pl.when(b + 1 < nblocks)
            def _():
                _ed_wait(1 - eb)
                pltpu.async_copy(h_sh.at[sidx_v.at[1 - eb, 0]],
                                 rows_v.at[1 - buf], sem_g)

        @pl.when(b * K + g >= 2)
        def _():
            pltpu.make_async_copy(srows_v.at[buf],
                                  acc_sh.at[didx_v.at[eb, 0]], sem_s).wait()

        _scale(buf, eb, g)
        pltpu.async_copy(srows_v.at[buf], acc_sh.at[didx_v.at[eb, g]],
                         sem_s, add=True)

    def _block(b, eb):
        _iter(b, 0, 0, eb, False)
        _iter(b, 1, 1, eb, False)

        @pl.when(b + 1 < nblocks)
        def _():
            _ed_copy(b + 1, 1 - eb)

        def _pair(p, carry):
            _iter(b, 2 * (p + 1), 0, eb, False)
            _iter(b, 2 * (p + 1) + 1, 1, eb, False)
            return carry

        lax.fori_loop(0, K // 2 - 2, _pair, 0)
        _iter(b, K - 2, 0, eb, False)
        _iter(b, K - 1, 1, eb, True)

    _ed_copy(0, 0)
    _ed_wait(0)
    pltpu.async_copy(h_sh.at[sidx_v.at[0, 0]], rows_v.at[0], sem_g)

    def _bpair(q, carry):
        _block(2 * q, 0)
        _block(2 * q + 1, 1)
        return carry

    lax.fori_loop(0, nblocks // 2, _bpair, 0)
    pltpu.make_async_copy(srows_v.at[0], acc_sh.at[didx_v.at[0, 0]], sem_s).wait()
    pltpu.make_async_copy(srows_v.at[1], acc_sh.at[didx_v.at[0, 0]], sem_s).wait()
    plsc.subcore_barrier()

    for k in range(ROWS_PER_TILE // CHUNK):
        r0 = s * ROWS_PER_TILE + k * CHUNK
        pltpu.sync_copy(acc_sh.at[pl.ds(r0, CHUNK)], rows_v.at[k % 2])
        pltpu.sync_copy(rows_v.at[k % 2],
                        out_hbm.at[pl.ds(r0, CHUNK), pl.ds(c * DH, DH)])


def _aggregate(hsplit, src, dst, ew, nblocks):
    mesh = plsc.VectorSubcoreMesh(core_axis_name="c", subcore_axis_name="s")
    body = functools.partial(_agg_body, nblocks=nblocks)
    return pl.kernel(
        body,
        out_type=jax.ShapeDtypeStruct((N_PAD, D), jnp.float32),
        mesh=mesh,
        compiler_params=pltpu.CompilerParams(use_tc_tiling_on_sc=False),
        scratch_types=[
            pltpu.VMEM((2, K, CHUNK), jnp.int32),
            pltpu.VMEM((2, K, CHUNK), jnp.int32),
            pltpu.VMEM((2, K, CHUNK), jnp.float32),
            pltpu.VMEM((2, CHUNK, DH), jnp.float32),
            pltpu.VMEM((2, CHUNK, DH), jnp.float32),
            pltpu.VMEM_SHARED((N_PAD, DH), jnp.float32),
            pltpu.VMEM_SHARED((N_PAD, DH), jnp.float32),
            pltpu.SemaphoreType.DMA,
            pltpu.SemaphoreType.DMA,
            pltpu.SemaphoreType.DMA,
        ],
    )(hsplit, src, dst, ew)


def kernel(x, edge_index, edge_weight, W, b):
    n_edges = edge_index.shape[1]
    src = edge_index[1].astype(jnp.int32)
    dst = edge_index[0].astype(jnp.int32)
    w = edge_weight.astype(jnp.float32)

    quantum = NS * CHUNK * K * 2
    e_pad = ((n_edges + quantum - 1) // quantum) * quantum
    if e_pad != n_edges:
        pad = e_pad - n_edges
        src = jnp.concatenate([src, jnp.zeros((pad,), jnp.int32)])
        dst = jnp.concatenate([dst, jnp.zeros((pad,), jnp.int32)])
        w = jnp.concatenate([w, jnp.zeros((pad,), jnp.float32)])
    nblocks = e_pad // (NS * CHUNK * K)

    src2 = src.reshape(-1, CHUNK)
    dst2 = dst.reshape(-1, CHUNK)
    ew2 = w.reshape(-1, CHUNK)

    x_pad = jnp.concatenate(
        [x, jnp.zeros((N_PAD - x.shape[0], D), jnp.float32)])
    hsplit = _matmul(x_pad, W, b)
    out = _aggregate(hsplit, src2, dst2, ew2, nblocks)
    return out[:N_NODES]


# speedup vs baseline: 1.0136x; 1.0136x over previous
import functools

import jax
import jax.numpy as jnp
from jax import lax
from jax.experimental import pallas as pl
from jax.experimental.pallas import tpu as pltpu
from jax.experimental.pallas import tpu_sc as plsc

N_NODES = 10000
D = 128
DH = D // 2
N_PAD = 10240
NC, NS, L = 2, 16, 16
CHUNK = 128
ROWS_PER_TILE = N_PAD // NS
K = 16


def _mm_body(x_ref, w_ref, b_ref, h_ref):
    h = (
        jnp.dot(x_ref[...], w_ref[...], preferred_element_type=jnp.float32)
        + b_ref[...]
    )
    h_ref[0, ...] = h[:, :DH]
    h_ref[1, ...] = h[:, DH:]


def _matmul(x, W, b):
    M = x.shape[0]
    BM = 1000
    return pl.pallas_call(
        _mm_body,
        grid=(M // BM,),
        in_specs=[
            pl.BlockSpec((BM, D), lambda i: (i, 0)),
            pl.BlockSpec((D, D), lambda i: (0, 0)),
            pl.BlockSpec((1, D), lambda i: (0, 0)),
        ],
        out_specs=pl.BlockSpec((2, BM, DH), lambda i: (0, i, 0)),
        out_shape=jax.ShapeDtypeStruct((2, N_PAD, DH), jnp.float32),
    )(x, W, b.reshape(1, D))


def _agg_body(h_hbm, src_hbm, dst_hbm, ew_hbm, out_hbm,
              sidx_v, didx_v, ew_v, rows_v, srows_v, h_sh, acc_sh,
              sem_g, sem_s, sem_e, nblocks):
    c = lax.axis_index("c")
    s = lax.axis_index("s")

    zero = jnp.zeros((L,), jnp.float32)

    def _zrow(i, carry):
        for j in range(DH // L):
            rows_v[0, i, pl.ds(j * L, L)] = zero
        return carry

    lax.fori_loop(0, CHUNK, _zrow, 0)
    for k in range(ROWS_PER_TILE // CHUNK):
        r0 = s * ROWS_PER_TILE + k * CHUNK
        pltpu.sync_copy(rows_v.at[0], acc_sh.at[pl.ds(r0, CHUNK)])
        pltpu.sync_copy(h_hbm.at[c, pl.ds(r0, CHUNK)], rows_v.at[1])
        pltpu.sync_copy(rows_v.at[1], h_sh.at[pl.ds(r0, CHUNK)])
    plsc.subcore_barrier()

    def _ed_copy(b, eb):
        base = s * nblocks + b
        pltpu.async_copy(src_hbm.at[pl.ds(base * K, K)], sidx_v.at[eb], sem_e)
        pltpu.async_copy(dst_hbm.at[pl.ds(base * K, K)], didx_v.at[eb], sem_e)
        pltpu.async_copy(ew_hbm.at[pl.ds(base * K, K)], ew_v.at[eb], sem_e)

    def _ed_wait(eb):
        pltpu.make_async_copy(src_hbm.at[pl.ds(0, K)], sidx_v.at[eb], sem_e).wait()
        pltpu.make_async_copy(dst_hbm.at[pl.ds(0, K)], didx_v.at[eb], sem_e).wait()
        pltpu.make_async_copy(ew_hbm.at[pl.ds(0, K)], ew_v.at[eb], sem_e).wait()

    def _scale(buf, eb, g):
        rb = rows_v.at[buf]
        sb = srows_v.at[buf]

        def _grp(i, carry):
            w16 = ew_v[eb, g, pl.ds(i * L, L)]
            for ii in range(L):
                e = i * L + ii
                w = w16[ii]
                for j in range(DH // L):
                    sb[e, pl.ds(j * L, L)] = rb[e, pl.ds(j * L, L)] * w
            return carry

        lax.fori_loop(0, CHUNK // L, _grp, 0)

    def _iter(b, g, buf, eb, last):
        pltpu.make_async_copy(h_sh.at[sidx_v.at[eb, 0]],
                              rows_v.at[buf], sem_g).wait()

        if not last:
            pltpu.async_copy(h_sh.at[sidx_v.at[eb, g + 1]],
                             rows_v.at[1 - buf], sem_g)
        else:
            @pl.when(b + 1 < nblocks)
            def _():
                _ed_wait(1 - eb)
                pltpu.async_copy(h_sh.at[sidx_v.at[1 - eb, 0]],
                                 rows_v.at[1 - buf], sem_g)

        @pl.when(b * K + g >= 2)
        def _():
            pltpu.make_async_copy(srows_v.at[buf],
                                  acc_sh.at[didx_v.at[eb, 0]], sem_s).wait()

        _scale(buf, eb, g)
        pltpu.async_copy(srows_v.at[buf], acc_sh.at[didx_v.at[eb, g]],
                         sem_s, add=True)

    def _block(b, eb):
        _iter(b, 0, 0, eb, False)
        _iter(b, 1, 1, eb, False)

        @pl.when(b + 1 < nblocks)
        def _():
            _ed_copy(b + 1, 1 - eb)

        def _pair(p, carry):
            _iter(b, 2 * (p + 1), 0, eb, False)
            _iter(b, 2 * (p + 1) + 1, 1, eb, False)
            return carry

        lax.fori_loop(0, K // 2 - 2, _pair, 0)
        _iter(b, K - 2, 0, eb, False)
        _iter(b, K - 1, 1, eb, True)

    _ed_copy(0, 0)
    _ed_wait(0)
    pltpu.async_copy(h_sh.at[sidx_v.at[0, 0]], rows_v.at[0], sem_g)

    def _bpair(q, carry):
        _block(2 * q, 0)
        _block(2 * q + 1, 1)
        return carry

    lax.fori_loop(0, nblocks // 2, _bpair, 0)
    pltpu.make_async_copy(srows_v.at[0], acc_sh.at[didx_v.at[0, 0]], sem_s).wait()
    pltpu.make_async_copy(srows_v.at[1], acc_sh.at[didx_v.at[0, 0]], sem_s).wait()
    plsc.subcore_barrier()

    for k in range(ROWS_PER_TILE // CHUNK):
        r0 = s * ROWS_PER_TILE + k * CHUNK
        pltpu.sync_copy(acc_sh.at[pl.ds(r0, CHUNK)], rows_v.at[k % 2])
        pltpu.sync_copy(rows_v.at[k % 2],
                        out_hbm.at[pl.ds(r0, CHUNK), pl.ds(c * DH, DH)])


def _aggregate(hsplit, src, dst, ew, nblocks):
    mesh = plsc.VectorSubcoreMesh(core_axis_name="c", subcore_axis_name="s")
    body = functools.partial(_agg_body, nblocks=nblocks)
    return pl.kernel(
        body,
        out_type=jax.ShapeDtypeStruct((N_PAD, D), jnp.float32),
        mesh=mesh,
        compiler_params=pltpu.CompilerParams(use_tc_tiling_on_sc=False),
        scratch_types=[
            pltpu.VMEM((2, K, CHUNK), jnp.int32),
            pltpu.VMEM((2, K, CHUNK), jnp.int32),
            pltpu.VMEM((2, K, CHUNK), jnp.float32),
            pltpu.VMEM((2, CHUNK, DH), jnp.float32),
            pltpu.VMEM((2, CHUNK, DH), jnp.float32),
            pltpu.VMEM_SHARED((N_PAD, DH), jnp.float32),
            pltpu.VMEM_SHARED((N_PAD, DH), jnp.float32),
            pltpu.SemaphoreType.DMA,
            pltpu.SemaphoreType.DMA,
            pltpu.SemaphoreType.DMA,
        ],
    )(hsplit, src, dst, ew)


def kernel(x, edge_index, edge_weight, W, b):
    n_edges = edge_index.shape[1]
    src = edge_index[1].astype(jnp.int32)
    dst = edge_index[0].astype(jnp.int32)
    w = edge_weight.astype(jnp.float32)

    quantum = NS * CHUNK * K * 2
    e_pad = ((n_edges + quantum - 1) // quantum) * quantum
    if e_pad != n_edges:
        pad = e_pad - n_edges
        src = jnp.concatenate([src, jnp.zeros((pad,), jnp.int32)])
        dst = jnp.concatenate([dst, jnp.zeros((pad,), jnp.int32)])
        w = jnp.concatenate([w, jnp.zeros((pad,), jnp.float32)])
    nblocks = e_pad // (NS * CHUNK * K)

    src2 = src.reshape(-1, CHUNK)
    dst2 = dst.reshape(-1, CHUNK)
    ew2 = w.reshape(-1, CHUNK)

    hsplit = _matmul(x, W, b)
    out = _aggregate(hsplit, src2, dst2, ew2, nblocks)
    return out[:N_NODES]


# speedup vs baseline: 1.0323x; 1.0185x over previous
import functools

import jax
import jax.numpy as jnp
from jax import lax
from jax.experimental import pallas as pl
from jax.experimental.pallas import tpu as pltpu
from jax.experimental.pallas import tpu_sc as plsc

N_NODES = 10000
D = 128
DH = D // 2
N_PAD = 10240
NC, NS, L = 2, 16, 16
CHUNK = 128
ROWS_PER_TILE = N_PAD // NS
K = 16


def _mm_body(x_ref, w_ref, b_ref, h_ref):
    h = (
        jnp.dot(x_ref[...], w_ref[...], preferred_element_type=jnp.float32)
        + b_ref[...]
    )
    h_ref[0, ...] = h[:, :DH]
    h_ref[1, ...] = h[:, DH:]


def _matmul(x, W, b):
    M = x.shape[0]
    BM = 1000
    return pl.pallas_call(
        _mm_body,
        grid=(M // BM,),
        in_specs=[
            pl.BlockSpec((BM, D), lambda i: (i, 0)),
            pl.BlockSpec((D, D), lambda i: (0, 0)),
            pl.BlockSpec((1, D), lambda i: (0, 0)),
        ],
        out_specs=pl.BlockSpec((2, BM, DH), lambda i: (0, i, 0)),
        out_shape=jax.ShapeDtypeStruct((2, N_PAD, DH), jnp.float32),
    )(x, W, b.reshape(1, D))


def _agg_body(h_hbm, src_hbm, dst_hbm, ew_hbm, out_hbm,
              sidx_v, didx_v, ew_v, rows_v, srows_v, h_sh, acc_sh,
              sem_g, sem_s, sem_e, nblocks):
    c = lax.axis_index("c")
    s = lax.axis_index("s")

    zero = jnp.zeros((L,), jnp.float32)

    def _zrow(i, carry):
        for j in range(DH // L):
            srows_v[0, i, pl.ds(j * L, L)] = zero
        return carry

    lax.fori_loop(0, CHUNK, _zrow, 0)
    NST = ROWS_PER_TILE // CHUNK

    def _r0(k):
        return s * ROWS_PER_TILE + k * CHUNK

    for k in range(NST):
        pltpu.async_copy(srows_v.at[0], acc_sh.at[pl.ds(_r0(k), CHUNK)], sem_s)
    pltpu.async_copy(h_hbm.at[c, pl.ds(_r0(0), CHUNK)], rows_v.at[0], sem_g)
    pltpu.async_copy(h_hbm.at[c, pl.ds(_r0(1), CHUNK)], rows_v.at[1], sem_g)
    for k in range(NST):
        pltpu.make_async_copy(h_hbm.at[c, pl.ds(_r0(k), CHUNK)],
                              rows_v.at[k % 2], sem_g).wait()
        pltpu.async_copy(rows_v.at[k % 2], h_sh.at[pl.ds(_r0(k), CHUNK)], sem_e)
        if k + 2 < NST:
            pltpu.make_async_copy(rows_v.at[k % 2],
                                  h_sh.at[pl.ds(_r0(k), CHUNK)], sem_e).wait()
            pltpu.async_copy(h_hbm.at[c, pl.ds(_r0(k + 2), CHUNK)],
                             rows_v.at[k % 2], sem_g)
    for k in range(2):
        pltpu.make_async_copy(rows_v.at[k], h_sh.at[pl.ds(_r0(0), CHUNK)],
                              sem_e).wait()
    for k in range(NST):
        pltpu.make_async_copy(srows_v.at[0], acc_sh.at[pl.ds(_r0(k), CHUNK)],
                              sem_s).wait()
    plsc.subcore_barrier()

    def _ed_copy(b, eb):
        base = s * nblocks + b
        pltpu.async_copy(src_hbm.at[pl.ds(base * K, K)], sidx_v.at[eb], sem_e)
        pltpu.async_copy(dst_hbm.at[pl.ds(base * K, K)], didx_v.at[eb], sem_e)
        pltpu.async_copy(ew_hbm.at[pl.ds(base * K, K)], ew_v.at[eb], sem_e)

    def _ed_wait(eb):
        pltpu.make_async_copy(src_hbm.at[pl.ds(0, K)], sidx_v.at[eb], sem_e).wait()
        pltpu.make_async_copy(dst_hbm.at[pl.ds(0, K)], didx_v.at[eb], sem_e).wait()
        pltpu.make_async_copy(ew_hbm.at[pl.ds(0, K)], ew_v.at[eb], sem_e).wait()

    def _scale(buf, eb, g):
        rb = rows_v.at[buf]
        sb = srows_v.at[buf]

        def _grp(i, carry):
            w16 = ew_v[eb, g, pl.ds(i * L, L)]
            for ii in range(L):
                e = i * L + ii
                w = w16[ii]
                for j in range(DH // L):
                    sb[e, pl.ds(j * L, L)] = rb[e, pl.ds(j * L, L)] * w
            return carry

        lax.fori_loop(0, CHUNK // L, _grp, 0)

    def _iter(b, g, buf, eb, last):
        pltpu.make_async_copy(h_sh.at[sidx_v.at[eb, 0]],
                              rows_v.at[buf], sem_g).wait()

        if not last:
            pltpu.async_copy(h_sh.at[sidx_v.at[eb, g + 1]],
                             rows_v.at[1 - buf], sem_g)
        else:
            @pl.when(b + 1 < nblocks)
            def _():
                _ed_wait(1 - eb)
                pltpu.async_copy(h_sh.at[sidx_v.at[1 - eb, 0]],
                                 rows_v.at[1 - buf], sem_g)

        @pl.when(b * K + g >= 2)
        def _():
            pltpu.make_async_copy(srows_v.at[buf],
                                  acc_sh.at[didx_v.at[eb, 0]], sem_s).wait()

        _scale(buf, eb, g)
        pltpu.async_copy(srows_v.at[buf], acc_sh.at[didx_v.at[eb, g]],
                         sem_s, add=True)

    def _block(b, eb):
        _iter(b, 0, 0, eb, False)
        _iter(b, 1, 1, eb, False)

        @pl.when(b + 1 < nblocks)
        def _():
            _ed_copy(b + 1, 1 - eb)

        def _pair(p, carry):
            _iter(b, 2 * (p + 1), 0, eb, False)
            _iter(b, 2 * (p + 1) + 1, 1, eb, False)
            return carry

        lax.fori_loop(0, K // 2 - 2, _pair, 0)
        _iter(b, K - 2, 0, eb, False)
        _iter(b, K - 1, 1, eb, True)

    _ed_copy(0, 0)
    _ed_wait(0)
    pltpu.async_copy(h_sh.at[sidx_v.at[0, 0]], rows_v.at[0], sem_g)

    def _bpair(q, carry):
        _block(2 * q, 0)
        _block(2 * q + 1, 1)
        return carry

    lax.fori_loop(0, nblocks // 2, _bpair, 0)
    pltpu.make_async_copy(srows_v.at[0], acc_sh.at[didx_v.at[0, 0]], sem_s).wait()
    pltpu.make_async_copy(srows_v.at[1], acc_sh.at[didx_v.at[0, 0]], sem_s).wait()
    plsc.subcore_barrier()

    for k in range(ROWS_PER_TILE // CHUNK):
        r0 = s * ROWS_PER_TILE + k * CHUNK
        pltpu.sync_copy(acc_sh.at[pl.ds(r0, CHUNK)], rows_v.at[k % 2])
        pltpu.sync_copy(rows_v.at[k % 2],
                        out_hbm.at[pl.ds(r0, CHUNK), pl.ds(c * DH, DH)])


def _aggregate(hsplit, src, dst, ew, nblocks):
    mesh = plsc.VectorSubcoreMesh(core_axis_name="c", subcore_axis_name="s")
    body = functools.partial(_agg_body, nblocks=nblocks)
    return pl.kernel(
        body,
        out_type=jax.ShapeDtypeStruct((N_PAD, D), jnp.float32),
        mesh=mesh,
        compiler_params=pltpu.CompilerParams(use_tc_tiling_on_sc=False),
        scratch_types=[
            pltpu.VMEM((2, K, CHUNK), jnp.int32),
            pltpu.VMEM((2, K, CHUNK), jnp.int32),
            pltpu.VMEM((2, K, CHUNK), jnp.float32),
            pltpu.VMEM((2, CHUNK, DH), jnp.float32),
            pltpu.VMEM((2, CHUNK, DH), jnp.float32),
            pltpu.VMEM_SHARED((N_PAD, DH), jnp.float32),
            pltpu.VMEM_SHARED((N_PAD, DH), jnp.float32),
            pltpu.SemaphoreType.DMA,
            pltpu.SemaphoreType.DMA,
            pltpu.SemaphoreType.DMA,
        ],
    )(hsplit, src, dst, ew)


def kernel(x, edge_index, edge_weight, W, b):
    n_edges = edge_index.shape[1]
    src = edge_index[1].astype(jnp.int32)
    dst = edge_index[0].astype(jnp.int32)
    w = edge_weight.astype(jnp.float32)

    quantum = NS * CHUNK * K * 2
    e_pad = ((n_edges + quantum - 1) // quantum) * quantum
    if e_pad != n_edges:
        pad = e_pad - n_edges
        src = jnp.concatenate([src, jnp.zeros((pad,), jnp.int32)])
        dst = jnp.concatenate([dst, jnp.zeros((pad,), jnp.int32)])
        w = jnp.concatenate([w, jnp.zeros((pad,), jnp.float32)])
    nblocks = e_pad // (NS * CHUNK * K)

    src2 = src.reshape(-1, CHUNK)
    dst2 = dst.reshape(-1, CHUNK)
    ew2 = w.reshape(-1, CHUNK)

    hsplit = _matmul(x, W, b)
    out = _aggregate(hsplit, src2, dst2, ew2, nblocks)
    return out[:N_NODES]


# speedup vs baseline: 1.0370x; 1.0045x over previous
import functools

import jax
import jax.numpy as jnp
from jax import lax
from jax.experimental import pallas as pl
from jax.experimental.pallas import tpu as pltpu
from jax.experimental.pallas import tpu_sc as plsc

N_NODES = 10000
D = 128
DH = D // 2
N_PAD = 10240
NC, NS, L = 2, 16, 16
CHUNK = 128
ROWS_PER_TILE = N_PAD // NS
K = 16


def _mm_body(x_ref, w_ref, b_ref, h_ref):
    h = (
        jnp.dot(x_ref[...], w_ref[...], preferred_element_type=jnp.float32)
        + b_ref[...]
    )
    h_ref[0, ...] = h[:, :DH]
    h_ref[1, ...] = h[:, DH:]


def _matmul(x, W, b):
    M = x.shape[0]
    BM = 1000
    return pl.pallas_call(
        _mm_body,
        grid=(M // BM,),
        in_specs=[
            pl.BlockSpec((BM, D), lambda i: (i, 0)),
            pl.BlockSpec((D, D), lambda i: (0, 0)),
            pl.BlockSpec((1, D), lambda i: (0, 0)),
        ],
        out_specs=pl.BlockSpec((2, BM, DH), lambda i: (0, i, 0)),
        out_shape=jax.ShapeDtypeStruct((2, N_PAD, DH), jnp.float32),
    )(x, W, b.reshape(1, D))


def _agg_body(h_hbm, src_hbm, dst_hbm, ew_hbm, out_hbm,
              sidx_v, didx_v, ew_v, rows_v, srows_v, h_sh, acc_sh,
              sem_g, sem_s, sem_e, nblocks):
    c = lax.axis_index("c")
    s = lax.axis_index("s")

    zero = jnp.zeros((L,), jnp.float32)

    def _zrow(i, carry):
        for j in range(DH // L):
            srows_v[0, i, pl.ds(j * L, L)] = zero
        return carry

    lax.fori_loop(0, CHUNK, _zrow, 0)
    NST = ROWS_PER_TILE // CHUNK

    def _r0(k):
        return s * ROWS_PER_TILE + k * CHUNK

    for k in range(NST):
        pltpu.async_copy(srows_v.at[0], acc_sh.at[pl.ds(_r0(k), CHUNK)], sem_s)
    pltpu.async_copy(h_hbm.at[c, pl.ds(_r0(0), CHUNK)], rows_v.at[0], sem_g)
    pltpu.async_copy(h_hbm.at[c, pl.ds(_r0(1), CHUNK)], rows_v.at[1], sem_g)
    for k in range(NST):
        pltpu.make_async_copy(h_hbm.at[c, pl.ds(_r0(k), CHUNK)],
                              rows_v.at[k % 2], sem_g).wait()
        pltpu.async_copy(rows_v.at[k % 2], h_sh.at[pl.ds(_r0(k), CHUNK)], sem_e)
        if k + 2 < NST:
            pltpu.make_async_copy(rows_v.at[k % 2],
                                  h_sh.at[pl.ds(_r0(k), CHUNK)], sem_e).wait()
            pltpu.async_copy(h_hbm.at[c, pl.ds(_r0(k + 2), CHUNK)],
                             rows_v.at[k % 2], sem_g)
    for k in range(2):
        pltpu.make_async_copy(rows_v.at[k], h_sh.at[pl.ds(_r0(0), CHUNK)],
                              sem_e).wait()
    for k in range(NST):
        pltpu.make_async_copy(srows_v.at[0], acc_sh.at[pl.ds(_r0(k), CHUNK)],
                              sem_s).wait()
    plsc.subcore_barrier()

    def _ed_copy(b, eb):
        base = s * nblocks + b
        pltpu.async_copy(src_hbm.at[pl.ds(base * K, K)], sidx_v.at[eb], sem_e)
        pltpu.async_copy(dst_hbm.at[pl.ds(base * K, K)], didx_v.at[eb], sem_e)
        pltpu.async_copy(ew_hbm.at[pl.ds(base * K, K)], ew_v.at[eb], sem_e)

    def _ed_wait(eb):
        pltpu.make_async_copy(src_hbm.at[pl.ds(0, K)], sidx_v.at[eb], sem_e).wait()
        pltpu.make_async_copy(dst_hbm.at[pl.ds(0, K)], didx_v.at[eb], sem_e).wait()
        pltpu.make_async_copy(ew_hbm.at[pl.ds(0, K)], ew_v.at[eb], sem_e).wait()

    def _scale(buf, eb, g):
        rb = rows_v.at[buf]
        sb = srows_v.at[buf]

        def _grp(i, carry):
            w16 = ew_v[eb, g, pl.ds(i * L, L)]
            for ii in range(L):
                e = i * L + ii
                w = w16[ii]
                for j in range(DH // L):
                    sb[e, pl.ds(j * L, L)] = rb[e, pl.ds(j * L, L)] * w
            return carry

        lax.fori_loop(0, CHUNK // L, _grp, 0)

    def _iter(b, g, buf, eb, last):
        pltpu.make_async_copy(h_sh.at[sidx_v.at[eb, 0]],
                              rows_v.at[buf], sem_g).wait()

        if not last:
            pltpu.async_copy(h_sh.at[sidx_v.at[eb, g + 1]],
                             rows_v.at[1 - buf], sem_g)
        else:
            @pl.when(b + 1 < nblocks)
            def _():
                _ed_wait(1 - eb)
                pltpu.async_copy(h_sh.at[sidx_v.at[1 - eb, 0]],
                                 rows_v.at[1 - buf], sem_g)

        @pl.when(b * K + g >= 2)
        def _():
            pltpu.make_async_copy(srows_v.at[buf],
                                  acc_sh.at[didx_v.at[eb, 0]], sem_s).wait()

        _scale(buf, eb, g)
        pltpu.async_copy(srows_v.at[buf], acc_sh.at[didx_v.at[eb, g]],
                         sem_s, add=True)

    def _block(b, eb):
        _iter(b, 0, 0, eb, False)
        _iter(b, 1, 1, eb, False)

        @pl.when(b + 1 < nblocks)
        def _():
            _ed_copy(b + 1, 1 - eb)

        def _pair(p, carry):
            _iter(b, 2 * (p + 1), 0, eb, False)
            _iter(b, 2 * (p + 1) + 1, 1, eb, False)
            return carry

        lax.fori_loop(0, K // 2 - 2, _pair, 0)
        _iter(b, K - 2, 0, eb, False)
        _iter(b, K - 1, 1, eb, True)

    _ed_copy(0, 0)
    _ed_wait(0)
    pltpu.async_copy(h_sh.at[sidx_v.at[0, 0]], rows_v.at[0], sem_g)

    def _bpair(q, carry):
        _block(2 * q, 0)
        _block(2 * q + 1, 1)
        return carry

    lax.fori_loop(0, nblocks // 2, _bpair, 0)
    pltpu.make_async_copy(srows_v.at[0], acc_sh.at[didx_v.at[0, 0]], sem_s).wait()
    pltpu.make_async_copy(srows_v.at[1], acc_sh.at[didx_v.at[0, 0]], sem_s).wait()
    plsc.subcore_barrier()

    pltpu.async_copy(acc_sh.at[pl.ds(_r0(0), CHUNK)], rows_v.at[0], sem_g)
    pltpu.async_copy(acc_sh.at[pl.ds(_r0(1), CHUNK)], rows_v.at[1], sem_g)
    for k in range(NST):
        pltpu.make_async_copy(acc_sh.at[pl.ds(_r0(k), CHUNK)],
                              rows_v.at[k % 2], sem_g).wait()
        pltpu.async_copy(rows_v.at[k % 2],
                         out_hbm.at[pl.ds(_r0(k), CHUNK), pl.ds(c * DH, DH)],
                         sem_e)
        if k + 2 < NST:
            pltpu.make_async_copy(
                rows_v.at[k % 2],
                out_hbm.at[pl.ds(_r0(k), CHUNK), pl.ds(c * DH, DH)],
                sem_e).wait()
            pltpu.async_copy(acc_sh.at[pl.ds(_r0(k + 2), CHUNK)],
                             rows_v.at[k % 2], sem_g)
    for k in range(2):
        pltpu.make_async_copy(
            rows_v.at[k],
            out_hbm.at[pl.ds(_r0(0), CHUNK), pl.ds(c * DH, DH)], sem_e).wait()


def _aggregate(hsplit, src, dst, ew, nblocks):
    mesh = plsc.VectorSubcoreMesh(core_axis_name="c", subcore_axis_name="s")
    body = functools.partial(_agg_body, nblocks=nblocks)
    return pl.kernel(
        body,
        out_type=jax.ShapeDtypeStruct((N_PAD, D), jnp.float32),
        mesh=mesh,
        compiler_params=pltpu.CompilerParams(use_tc_tiling_on_sc=False),
        scratch_types=[
            pltpu.VMEM((2, K, CHUNK), jnp.int32),
            pltpu.VMEM((2, K, CHUNK), jnp.int32),
            pltpu.VMEM((2, K, CHUNK), jnp.float32),
            pltpu.VMEM((2, CHUNK, DH), jnp.float32),
            pltpu.VMEM((2, CHUNK, DH), jnp.float32),
            pltpu.VMEM_SHARED((N_PAD, DH), jnp.float32),
            pltpu.VMEM_SHARED((N_PAD, DH), jnp.float32),
            pltpu.SemaphoreType.DMA,
            pltpu.SemaphoreType.DMA,
            pltpu.SemaphoreType.DMA,
        ],
    )(hsplit, src, dst, ew)


def kernel(x, edge_index, edge_weight, W, b):
    n_edges = edge_index.shape[1]
    src = edge_index[1].astype(jnp.int32)
    dst = edge_index[0].astype(jnp.int32)
    w = edge_weight.astype(jnp.float32)

    quantum = NS * CHUNK * K * 2
    e_pad = ((n_edges + quantum - 1) // quantum) * quantum
    if e_pad != n_edges:
        pad = e_pad - n_edges
        src = jnp.concatenate([src, jnp.zeros((pad,), jnp.int32)])
        dst = jnp.concatenate([dst, jnp.zeros((pad,), jnp.int32)])
        w = jnp.concatenate([w, jnp.zeros((pad,), jnp.float32)])
    nblocks = e_pad // (NS * CHUNK * K)

    src2 = src.reshape(-1, CHUNK)
    dst2 = dst.reshape(-1, CHUNK)
    ew2 = w.reshape(-1, CHUNK)

    hsplit = _matmul(x, W, b)
    out = _aggregate(hsplit, src2, dst2, ew2, nblocks)
    return out[:N_NODES]
